# Initial kernel scaffold; baseline (speedup 1.0000x reference)
import jax
import jax.numpy as jnp
from jax.experimental import pallas as pl


def kernel(edge_index, emb, W1, b1, W2, b2):
    raise NotImplementedError("write your pallas kernel here")


# speedup vs baseline: 10.6748x; 10.6748x over previous
import functools

import jax
import jax.numpy as jnp
from jax import lax
from jax.experimental import pallas as pl
from jax.experimental.pallas import tpu as pltpu
from jax.experimental.pallas import tpu_sc as plsc

N = 10000
NINP = 128
E = 320000

NC, NS = 2, 16
CHUNK = 128
ACC_ROWS = 10240
TRASH = N
DEG_W = 16

ROW_BLK = 400

_mesh = lambda: plsc.VectorSubcoreMesh(core_axis_name="c", subcore_axis_name="s")


def _edge_pass(nchunks, table_rows):
    @functools.partial(
        pl.kernel,
        mesh=_mesh(),
        out_type=jax.ShapeDtypeStruct((NC, ACC_ROWS, NINP), jnp.float32),
        scratch_types=[
            pltpu.VMEM((CHUNK,), jnp.int32),
            pltpu.VMEM((CHUNK,), jnp.int32),
            pltpu.VMEM((CHUNK, NINP), jnp.float32),
            pltpu.VMEM_SHARED((ACC_ROWS, NINP), jnp.float32),
            pltpu.SemaphoreType.DMA,
        ],
    )
    def k(src_hbm, dst_hbm, table_hbm, out_hbm, src_v, dst_v, rows_v, acc, sem):
        c = lax.axis_index("c")
        s = lax.axis_index("s")
        rows_per_tile = ACC_ROWS // NS

        def zrow(i, carry):
            for j in range(NINP // 16):
                rows_v[i, pl.ds(j * 16, 16)] = jnp.zeros((16,), jnp.float32)
            return carry
        lax.fori_loop(0, CHUNK, zrow, 0)
        for b in range(rows_per_tile // CHUNK):
            pltpu.sync_copy(
                rows_v, acc.at[pl.ds(s * rows_per_tile + b * CHUNK, CHUNK)])
        plsc.subcore_barrier()

        def body(j, carry):
            pltpu.sync_copy(src_hbm.at[c, s, j], src_v)
            pltpu.sync_copy(dst_hbm.at[c, s, j], dst_v)
            pltpu.async_copy(table_hbm.at[src_v], rows_v, sem).wait()
            pltpu.sync_copy(rows_v, acc.at[dst_v], add=True)
            return carry
        lax.fori_loop(0, nchunks, body, 0)

        plsc.subcore_barrier()
        pltpu.sync_copy(acc.at[pl.ds(s * rows_per_tile, rows_per_tile)],
                        out_hbm.at[c, pl.ds(s * rows_per_tile, rows_per_tile)])

    return k


def _deg_pass(nchunks):
    @functools.partial(
        pl.kernel,
        mesh=_mesh(),
        out_type=jax.ShapeDtypeStruct((NC, ACC_ROWS, NINP), jnp.float32),
        scratch_types=[
            pltpu.VMEM((CHUNK,), jnp.int32),
            pltpu.VMEM((CHUNK, NINP), jnp.float32),
            pltpu.VMEM_SHARED((ACC_ROWS, NINP), jnp.float32),
        ],
    )
    def k(dst_hbm, out_hbm, dst_v, ones_v, acc):
        c = lax.axis_index("c")
        s = lax.axis_index("s")
        rows_per_tile = ACC_ROWS // NS

        def fill(val):
            def frow(i, carry):
                for j in range(NINP // 16):
                    ones_v[i, pl.ds(j * 16, 16)] = jnp.full((16,), val,
                                                            jnp.float32)
                return carry
            lax.fori_loop(0, CHUNK, frow, 0)

        fill(0.0)
        for b in range(rows_per_tile // CHUNK):
            pltpu.sync_copy(
                ones_v, acc.at[pl.ds(s * rows_per_tile + b * CHUNK, CHUNK)])
        fill(1.0)
        plsc.subcore_barrier()

        def body(j, carry):
            pltpu.sync_copy(dst_hbm.at[c, s, j], dst_v)
            pltpu.sync_copy(ones_v, acc.at[dst_v], add=True)
            return carry
        lax.fori_loop(0, nchunks, body, 0)

        plsc.subcore_barrier()
        pltpu.sync_copy(acc.at[pl.ds(s * rows_per_tile, rows_per_tile)],
                        out_hbm.at[c, pl.ds(s * rows_per_tile, rows_per_tile)])

    return k


def _dis_of(degs_blk):
    deg = degs_blk[0, :, 0] + degs_blk[1, :, 0] + 1.0
    return lax.rsqrt(deg)[:, None]


def _tc_stage1(degs, emb, W1):
    def body(degs_ref, emb_ref, w_ref, g_ref):
        dis = _dis_of(degs_ref)
        h = jnp.dot(emb_ref[...], w_ref[...],
                    preferred_element_type=jnp.float32)
        g_ref[0] = h[:, :NINP] * dis
        g_ref[1] = h[:, NINP:] * dis

    return pl.pallas_call(
        body,
        grid=(N // ROW_BLK,),
        in_specs=[
            pl.BlockSpec((NC, ROW_BLK, DEG_W), lambda i: (0, i, 0)),
            pl.BlockSpec((ROW_BLK, NINP), lambda i: (i, 0)),
            pl.BlockSpec((NINP, 2 * NINP), lambda i: (0, 0)),
        ],
        out_specs=pl.BlockSpec((NC, ROW_BLK, NINP), lambda i: (0, i, 0)),
        out_shape=jax.ShapeDtypeStruct((NC, N, NINP), jnp.float32),
    )(degs, emb, W1)


def _tc_stage2(degs, s1, g1, W2, b1):
    def body(degs_ref, s1_ref, g1_ref, w_ref, b_ref, g2_ref):
        dis = _dis_of(degs_ref)
        t0 = (s1_ref[0] + g1_ref[0]) * dis
        t1 = (s1_ref[1] + g1_ref[1]) * dis
        x1 = jnp.concatenate([t0, t1], axis=1) + b_ref[...]
        g2_ref[...] = jnp.dot(x1, w_ref[...],
                              preferred_element_type=jnp.float32) * dis

    return pl.pallas_call(
        body,
        grid=(N // ROW_BLK,),
        in_specs=[
            pl.BlockSpec((NC, ROW_BLK, DEG_W), lambda i: (0, i, 0)),
            pl.BlockSpec((NC, ROW_BLK, NINP), lambda i: (0, i, 0)),
            pl.BlockSpec((NC, ROW_BLK, NINP), lambda i: (0, i, 0)),
            pl.BlockSpec((2 * NINP, NINP), lambda i: (0, 0)),
            pl.BlockSpec((1, 2 * NINP), lambda i: (0, 0)),
        ],
        out_specs=pl.BlockSpec((ROW_BLK, NINP), lambda i: (i, 0)),
        out_shape=jax.ShapeDtypeStruct((N, NINP), jnp.float32),
    )(degs, s1, g1, W2, b1)


def _tc_stage3(degs, s2, g2, b2):
    def body(degs_ref, s2_ref, g2_ref, b_ref, out_ref):
        dis = _dis_of(degs_ref)
        out_ref[...] = (s2_ref[0] + s2_ref[1] + g2_ref[...]) * dis + b_ref[...]

    return pl.pallas_call(
        body,
        grid=(N // ROW_BLK,),
        in_specs=[
            pl.BlockSpec((NC, ROW_BLK, DEG_W), lambda i: (0, i, 0)),
            pl.BlockSpec((NC, ROW_BLK, NINP), lambda i: (0, i, 0)),
            pl.BlockSpec((ROW_BLK, NINP), lambda i: (i, 0)),
            pl.BlockSpec((1, NINP), lambda i: (0, 0)),
        ],
        out_specs=pl.BlockSpec((ROW_BLK, NINP), lambda i: (i, 0)),
        out_shape=jax.ShapeDtypeStruct((N, NINP), jnp.float32),
    )(degs, s2, g2, b2)


def _ceil_div(a, b):
    return (a + b - 1) // b


@jax.jit
def kernel(edge_index, emb, W1, b1, W2, b2):
    src = edge_index[0].astype(jnp.int32)
    dst = edge_index[1].astype(jnp.int32)

    e_sub = E // NS
    c1 = _ceil_div(e_sub, CHUNK)
    pad1 = c1 * CHUNK - e_sub
    src1 = jnp.pad(src.reshape(NS, e_sub), ((0, 0), (0, pad1)))
    src1 = src1.reshape(NS, c1, CHUNK)
    dst1 = jnp.pad(dst.reshape(NS, e_sub), ((0, 0), (0, pad1)),
                   constant_values=TRASH).reshape(NS, c1, CHUNK)
    src_l1 = jnp.stack([src1, src1 + N])
    dst_l1 = jnp.stack([dst1, dst1])

    e_tile = E // (NC * NS)
    c2 = _ceil_div(e_tile, CHUNK)
    pad2 = c2 * CHUNK - e_tile
    src2 = jnp.pad(src.reshape(NC, NS, e_tile), ((0, 0), (0, 0), (0, pad2)))
    src2 = src2.reshape(NC, NS, c2, CHUNK)
    dst2 = jnp.pad(dst.reshape(NC, NS, e_tile), ((0, 0), (0, 0), (0, pad2)),
                   constant_values=TRASH).reshape(NC, NS, c2, CHUNK)

    degs = _deg_pass(c2)(dst2)[:, :, :DEG_W]
    g1 = _tc_stage1(degs, emb, W1)
    s1 = _edge_pass(c1, NC * N)(src_l1, dst_l1, g1.reshape(NC * N, NINP))
    g2 = _tc_stage2(degs, s1, g1, W2, b1.reshape(1, -1))
    s2 = _edge_pass(c2, N)(src2, dst2, g2)
    return _tc_stage3(degs, s2, g2, b2.reshape(1, -1))
